# Initial kernel scaffold; baseline (speedup 1.0000x reference)
import jax
import jax.numpy as jnp
from jax.experimental import pallas as pl


def kernel(x, lens, emb, W, b):
    raise NotImplementedError("write your pallas kernel here")


# speedup vs baseline: 11.3697x; 11.3697x over previous
import dataclasses
import functools

import jax
import jax.numpy as jnp
from jax import lax
from jax.experimental import pallas as pl
from jax.experimental.pallas import tpu as pltpu
from jax.experimental.pallas import tpu_sc as plsc

PAD_ID = 0
VOCAB = 100000
EMB = 128
OUT = 128
B, L = 16384, 200

GS = 2
TOK = GS * L
GCHUNK = 80
NCHUNK = TOK // GCHUNK
NVEC = EMB // 16
NSTEP = B // GS


def _pool_sc(x3, emb):
  mesh = plsc.VectorSubcoreMesh(core_axis_name="core", subcore_axis_name="subcore")
  cp = pltpu.CompilerParams()
  if "needs_layout_passes" in pltpu.CompilerParams.__dataclass_fields__:
    cp = dataclasses.replace(cp, needs_layout_passes=False)

  @functools.partial(
      pl.kernel,
      out_type=jax.ShapeDtypeStruct((NSTEP, GS, EMB), jnp.float32),
      mesh=mesh,
      compiler_params=cp,
      scratch_types=[
          pltpu.VMEM((TOK, EMB), jnp.float32),
          pltpu.VMEM((EMB,), jnp.float32),
          pltpu.SemaphoreType.DMA,
      ],
  )
  def pool(x_hbm, t_hbm, o_hbm, rows_v, emb0_v, sem):
    pltpu.sync_copy(t_hbm.at[0], emb0_v)
    lanes = lax.iota(jnp.int32, 16)
    lo_mask = lanes < 8

    def body(xv, ov):
      descs = [
          pltpu.async_copy(t_hbm.at[xv.at[0, j]], rows_v.at[pl.ds(j * GCHUNK, GCHUNK)], sem)
          for j in range(NCHUNK)
      ]
      cnt0 = jnp.zeros((16,), jnp.int32)
      cnt1 = jnp.zeros((16,), jnp.int32)
      zero = jnp.zeros((16,), jnp.int32)
      for j in range(NCHUNK):
        for m in range(GCHUNK // 16):
          t0 = j * GCHUNK + m * 16
          isz = jnp.where(xv[0, j, pl.ds(m * 16, 16)] == PAD_ID, 1, 0)
          if t0 + 16 <= L:
            cnt0 = cnt0 + isz
          elif t0 >= L:
            cnt1 = cnt1 + isz
          else:
            cnt0 = cnt0 + jnp.where(lo_mask, isz, zero)
            cnt1 = cnt1 + jnp.where(lo_mask, zero, isz)
      c0 = jnp.sum(cnt0).astype(jnp.float32)
      c1 = jnp.sum(cnt1).astype(jnp.float32)
      for d in descs:
        d.wait()
      for g, cf in ((0, c0), (1, c1)):
        init = tuple(jnp.zeros((16,), jnp.float32) for _ in range(NVEC))

        def red(i, accs, g=g):
          t = g * L + i
          return tuple(
              accs[c] + rows_v[t, pl.ds(c * 16, 16)] for c in range(NVEC)
          )

        accs = lax.fori_loop(0, L, red, init)
        cv = jnp.full((16,), cf)
        for c in range(NVEC):
          ov[0, g, pl.ds(c * 16, 16)] = accs[c] - cv * emb0_v[pl.ds(c * 16, 16)]

    pltpu.emit_pipeline(
        body,
        grid=(NSTEP,),
        in_specs=[
            pl.BlockSpec((1, NCHUNK, GCHUNK), lambda i: (i, 0, 0)),
        ],
        out_specs=[
            pl.BlockSpec((1, GS, EMB), lambda i: (i, 0, 0)),
        ],
        core_axis_name=("core", "subcore"),
        dimension_semantics=(pltpu.PARALLEL,),
    )(x_hbm, o_hbm)

  return pool(x3, emb)


BLK = 1024


def _proj_kernel(s_ref, l_ref, w_ref, b_ref, o_ref):
  mean = s_ref[...] / l_ref[...]
  o_ref[...] = (
      lax.dot_general(
          mean, w_ref[...], (((1,), (1,)), ((), ())),
          preferred_element_type=jnp.float32,
      )
      + b_ref[...]
  )


def _proj_tc(summed, lens2, W, b2):
  return pl.pallas_call(
      _proj_kernel,
      grid=(B // BLK,),
      in_specs=[
          pl.BlockSpec((BLK, EMB), lambda i: (i, 0)),
          pl.BlockSpec((BLK, 1), lambda i: (i, 0)),
          pl.BlockSpec((OUT, EMB), lambda i: (0, 0)),
          pl.BlockSpec((1, OUT), lambda i: (0, 0)),
      ],
      out_specs=pl.BlockSpec((BLK, OUT), lambda i: (i, 0)),
      out_shape=jax.ShapeDtypeStruct((B, OUT), jnp.float32),
  )(summed, lens2, W, b2)


@jax.jit
def kernel(x, lens, emb, W, b):
  x3 = x.astype(jnp.int32).reshape(NSTEP, NCHUNK, GCHUNK)
  summed = _pool_sc(x3, emb).reshape(B, EMB)
  return _proj_tc(summed, lens.reshape(B, 1), W, b.reshape(1, OUT))


# speedup vs baseline: 20.8914x; 1.8375x over previous
import dataclasses
import functools

import jax
import jax.numpy as jnp
from jax import lax
from jax.experimental import pallas as pl
from jax.experimental.pallas import tpu as pltpu
from jax.experimental.pallas import tpu_sc as plsc

PAD_ID = 0
VOCAB = 100000
EMB = 128
OUT = 128
B, L = 16384, 200

GS = 2
TOK = GS * L
GCHUNK = 80
NCHUNK = TOK // GCHUNK
NVEC = EMB // 16
NSTEP = B // GS
NWORKER = 32
SPW = NSTEP // NWORKER


def _pool_sc(x3, emb):
  mesh = plsc.VectorSubcoreMesh(core_axis_name="core", subcore_axis_name="subcore")
  cp = pltpu.CompilerParams()
  if "needs_layout_passes" in pltpu.CompilerParams.__dataclass_fields__:
    cp = dataclasses.replace(cp, needs_layout_passes=False)

  @functools.partial(
      pl.kernel,
      out_type=jax.ShapeDtypeStruct((NSTEP, GS, EMB), jnp.float32),
      mesh=mesh,
      compiler_params=cp,
      scratch_types=[
          pltpu.VMEM((TOK,), jnp.int32),
          pltpu.VMEM((TOK,), jnp.int32),
          pltpu.VMEM((TOK, EMB), jnp.float32),
          pltpu.VMEM((TOK, EMB), jnp.float32),
          pltpu.VMEM((GS, EMB), jnp.float32),
          pltpu.VMEM((GS, EMB), jnp.float32),
          pltpu.VMEM((EMB,), jnp.float32),
          pltpu.SemaphoreType.DMA((2,)),
          pltpu.SemaphoreType.DMA((2,)),
          pltpu.SemaphoreType.DMA((2,)),
      ],
  )
  def pool(x_hbm, t_hbm, o_hbm, xv0, xv1, rows0, rows1, ov0, ov1, emb0_v,
           xsem, gsem, osem):
    xvs, rowss, ovs = (xv0, xv1), (rows0, rows1), (ov0, ov1)
    wid = lax.axis_index("subcore") * 2 + lax.axis_index("core")
    base = wid * SPW
    pltpu.sync_copy(t_hbm.at[0], emb0_v)
    lanes = lax.iota(jnp.int32, 16)
    lo_mask = lanes < 8

    def copy_x(s, b):
      pltpu.async_copy(
          x_hbm.at[pl.ds((base + s) * TOK, TOK)], xvs[b], xsem.at[b])

    def wait_x(b):
      pltpu.make_async_copy(
          x_hbm.at[pl.ds(base * TOK, TOK)], xvs[b], xsem.at[b]).wait()

    def fire_gathers(b):
      for j in range(NCHUNK):
        pltpu.async_copy(
            t_hbm.at[xvs[b].at[pl.ds(j * GCHUNK, GCHUNK)]],
            rowss[b].at[pl.ds(j * GCHUNK, GCHUNK)],
            gsem.at[b],
        )

    def wait_gathers(b):
      for j in range(NCHUNK):
        pltpu.make_async_copy(
            t_hbm.at[xvs[b].at[pl.ds(j * GCHUNK, GCHUNK)]],
            rowss[b].at[pl.ds(j * GCHUNK, GCHUNK)],
            gsem.at[b],
        ).wait()

    def copy_out(s, b):
      pltpu.async_copy(ovs[b], o_hbm.at[base + s], osem.at[b])

    def wait_out(b):
      pltpu.make_async_copy(ovs[b], o_hbm.at[base], osem.at[b]).wait()

    def count_zeros(b):
      cnt0 = jnp.zeros((16,), jnp.int32)
      cnt1 = jnp.zeros((16,), jnp.int32)
      zero = jnp.zeros((16,), jnp.int32)
      for t0 in range(0, TOK, 16):
        isz = jnp.where(xvs[b][pl.ds(t0, 16)] == PAD_ID, 1, 0)
        if t0 + 16 <= L:
          cnt0 = cnt0 + isz
        elif t0 >= L:
          cnt1 = cnt1 + isz
        else:
          cnt0 = cnt0 + jnp.where(lo_mask, isz, zero)
          cnt1 = cnt1 + jnp.where(lo_mask, zero, isz)
      return jnp.sum(cnt0).astype(jnp.float32), jnp.sum(cnt1).astype(jnp.float32)

    def reduce_store(b, c0, c1):
      for g, cf in ((0, c0), (1, c1)):
        init = tuple(jnp.zeros((16,), jnp.float32) for _ in range(NVEC))

        def red(i, accs, g=g):
          t = g * L + i
          return tuple(
              accs[c] + rowss[b][t, pl.ds(c * 16, 16)] for c in range(NVEC)
          )

        accs = lax.fori_loop(0, L, red, init)
        cv = jnp.full((16,), cf)
        for c in range(NVEC):
          ovs[b][g, pl.ds(c * 16, 16)] = accs[c] - cv * emb0_v[pl.ds(c * 16, 16)]

    def step(s, b, fire, drain_out):
      nb = 1 - b
      if fire:
        wait_x(nb)
        fire_gathers(nb)
      c0, c1 = count_zeros(b)
      wait_gathers(b)
      if fire:
        copy_x(s + 2, b)
      if drain_out:
        wait_out(b)
      reduce_store(b, c0, c1)
      copy_out(s, b)

    pltpu.sync_copy(x_hbm.at[pl.ds(base * TOK, TOK)], xv0)
    fire_gathers(0)
    copy_x(1, 1)
    step(0, 0, True, False)
    step(1, 1, True, False)

    def loop_body(k, _):
      step(2 * k, 0, True, True)
      step(2 * k + 1, 1, True, True)
      return 0

    lax.fori_loop(1, SPW // 2 - 1, loop_body, 0)

    nb = 1
    wait_x(nb)
    fire_gathers(nb)
    c0, c1 = count_zeros(0)
    wait_gathers(0)
    wait_out(0)
    reduce_store(0, c0, c1)
    copy_out(SPW - 2, 0)
    step(SPW - 1, 1, False, True)
    wait_out(0)
    wait_out(1)

  return pool(x3, emb)


BLK = 1024


def _proj_kernel(s_ref, l_ref, w_ref, b_ref, o_ref):
  mean = s_ref[...] / l_ref[...]
  o_ref[...] = (
      lax.dot_general(
          mean, w_ref[...], (((1,), (1,)), ((), ())),
          preferred_element_type=jnp.float32,
      )
      + b_ref[...]
  )


def _proj_tc(summed, lens2, W, b2):
  return pl.pallas_call(
      _proj_kernel,
      grid=(B // BLK,),
      in_specs=[
          pl.BlockSpec((BLK, EMB), lambda i: (i, 0)),
          pl.BlockSpec((BLK, 1), lambda i: (i, 0)),
          pl.BlockSpec((OUT, EMB), lambda i: (0, 0)),
          pl.BlockSpec((1, OUT), lambda i: (0, 0)),
      ],
      out_specs=pl.BlockSpec((BLK, OUT), lambda i: (i, 0)),
      out_shape=jax.ShapeDtypeStruct((B, OUT), jnp.float32),
  )(summed, lens2, W, b2)


@jax.jit
def kernel(x, lens, emb, W, b):
  x3 = x.astype(jnp.int32).reshape(B * L)
  summed = _pool_sc(x3, emb).reshape(B, EMB)
  return _proj_tc(summed, lens.reshape(B, 1), W, b.reshape(1, OUT))
